# baseline (speedup 1.0000x reference)
import functools

import jax
import jax.numpy as jnp
from jax import lax
from jax.experimental import pallas as pl
from jax.experimental.pallas import tpu as pltpu
from jax.experimental.pallas import tpu_sc as plsc

N = 10000
E = 320000
D = 128
L = 16
NC = 2
NS = 16
NW = NC * NS
EPT = E // NW
C = 80
NCHUNK = EPT // C
ZR = C
ZCH = N // ZR
ZROUNDS = -(-ZCH // NS)
assert EPT % C == 0 and N % ZR == 0 and C % 8 == 0 and ZR % 8 == 0


def _sc_aggregate(src, ew, feat):
    mesh = plsc.VectorSubcoreMesh(
        core_axis_name="c", subcore_axis_name="s", num_cores=NC, num_subcores=NS
    )

    @functools.partial(
        pl.kernel,
        out_type=jax.ShapeDtypeStruct((NC * N, D), jnp.float32),
        mesh=mesh,
        scratch_types=[
            pltpu.VMEM((1, EPT), jnp.int32),
            pltpu.VMEM((1, EPT), jnp.int32),
            pltpu.VMEM((C, D), jnp.float32),
            pltpu.VMEM((C, D), jnp.float32),
            pltpu.VMEM((C, D), jnp.float32),
            pltpu.VMEM((C,), jnp.float32),
            pltpu.VMEM((C,), jnp.float32),
            pltpu.VMEM((C,), jnp.float32),
            pltpu.VMEM_SHARED((N, D), jnp.float32),
            [pltpu.SemaphoreType.DMA] * 3,
            [pltpu.SemaphoreType.DMA] * 3,
            [pltpu.SemaphoreType.DMA] * 3,
        ],
        compiler_params=pltpu.CompilerParams(use_tc_tiling_on_sc=False),
    )
    def body(edge_hbm, ew_hbm, feat_hbm, out_hbm,
             src_v, dst_v, rows0_v, rows1_v, rows2_v,
             ewb0_v, ewb1_v, ewb2_v, acc_sh, gsems, esems, ssems):
        stage_v = rows0_v
        c = lax.axis_index("c")
        s = lax.axis_index("s")
        wid = s * NC + c

        rows = (rows0_v, rows1_v, rows2_v)
        ewbs = (ewb0_v, ewb1_v, ewb2_v)

        pltpu.sync_copy(edge_hbm.at[pl.ds(0, 1), pl.ds(wid * EPT, EPT)], src_v)
        pltpu.sync_copy(edge_hbm.at[pl.ds(1, 1), pl.ds(wid * EPT, EPT)], dst_v)

        @plsc.parallel_loop(0, ZR, 1, unroll=4)
        def zero_row(i):
            for j in range(D // L):
                stage_v[i, pl.ds(j * L, L)] = jnp.zeros((L,), jnp.float32)

        for k in range(ZROUNDS):
            i = s + k * NS

            @pl.when(i < ZCH)
            def _():
                pltpu.async_copy(stage_v, acc_sh.at[pl.ds(i * ZR, ZR)],
                                 gsems[0])

        for k in range(ZROUNDS):
            i = s + k * NS

            @pl.when(i < ZCH)
            def _():
                pltpu.make_async_copy(
                    stage_v, acc_sh.at[pl.ds(i * ZR, ZR)], gsems[0]
                ).wait()

        plsc.subcore_barrier()

        dnums = lax.GatherDimensionNumbers(
            offset_dims=(), collapsed_slice_dims=(0,), start_index_map=(0,)
        )

        def gather_start(ci, b):
            pltpu.make_async_copy(
                feat_hbm.at[src_v.at[0, pl.ds(ci * C, C)]], rows[b], gsems[b]
            ).start()
            pltpu.async_copy(ew_hbm.at[pl.ds(wid * EPT + ci * C, C)],
                             ewbs[b], esems[b])

        def gather_wait(ci, b):
            pltpu.make_async_copy(
                feat_hbm.at[src_v.at[0, pl.ds(ci * C, C)]], rows[b], gsems[b]
            ).wait()
            pltpu.make_async_copy(
                ew_hbm.at[pl.ds(wid * EPT + ci * C, C)], ewbs[b], esems[b]
            ).wait()

        def scatter_start(ci, b):
            pltpu.async_copy(rows[b], acc_sh.at[dst_v.at[0, pl.ds(ci * C, C)]],
                             ssems[b], add=True)

        def scatter_wait(ci, b):
            pltpu.make_async_copy(
                rows[b], acc_sh.at[dst_v.at[0, pl.ds(ci * C, C)]], ssems[b]
            ).wait()

        def scale(ci, b):
            buf, ewb = rows[b], ewbs[b]

            @plsc.parallel_loop(0, C // L, 1, unroll=2)
            def group(g):
                wvec = ewb[pl.ds(g * L, L)]
                for ll in range(L):
                    e = g * L + ll
                    w = lax.gather(
                        wvec,
                        jnp.full((L, 1), ll, jnp.int32),
                        dnums,
                        slice_sizes=(1,),
                        mode=lax.GatherScatterMode.PROMISE_IN_BOUNDS,
                    )
                    for j in range(D // L):
                        sl = pl.ds(j * L, L)
                        buf[e, sl] = buf[e, sl] * w

        gather_start(0, 0)
        gather_start(1, 1)
        gather_wait(0, 0)
        scale(0, 0)
        gather_start(2, 2)
        scatter_start(0, 0)
        gather_wait(1, 1)
        scale(1, 1)
        scatter_start(1, 1)

        def chunk_triple(o, carry):
            c0 = 3 * o + 2
            scatter_wait(c0 - 2, 0)
            gather_start(c0 + 1, 0)
            gather_wait(c0, 2)
            scale(c0, 2)
            scatter_start(c0, 2)

            scatter_wait(c0 - 1, 1)
            gather_start(c0 + 2, 1)
            gather_wait(c0 + 1, 0)
            scale(c0 + 1, 0)
            scatter_start(c0 + 1, 0)

            scatter_wait(c0, 2)

            @pl.when(c0 + 3 < NCHUNK)
            def _():
                gather_start(c0 + 3, 2)

            gather_wait(c0 + 2, 1)
            scale(c0 + 2, 1)
            scatter_start(c0 + 2, 1)
            return carry

        lax.fori_loop(0, (NCHUNK - 2) // 3, chunk_triple, 0)
        scatter_wait(NCHUNK - 2, 0)
        scatter_wait(NCHUNK - 1, 1)
        plsc.subcore_barrier()

        for k in range(ZROUNDS):
            b = k % 3
            i = s + k * NS

            @pl.when(i < ZCH)
            def _():
                if k >= 3:
                    ip = s + (k - 3) * NS
                    pltpu.make_async_copy(
                        rows[b], out_hbm.at[pl.ds(c * N + ip * ZR, ZR)],
                        ssems[b],
                    ).wait()
                pltpu.sync_copy(acc_sh.at[pl.ds(i * ZR, ZR)], rows[b])
                pltpu.async_copy(
                    rows[b], out_hbm.at[pl.ds(c * N + i * ZR, ZR)], ssems[b]
                )

        for k in range(ZROUNDS - 3, ZROUNDS):
            b = k % 3
            i = s + k * NS

            @pl.when(i < ZCH)
            def _():
                pltpu.make_async_copy(
                    rows[b], out_hbm.at[pl.ds(c * N + i * ZR, ZR)], ssems[b]
                ).wait()

    return body(src, ew, feat)


BN = 1000


def _tc_matmul(parts, weight, bias2d):
    def body(p_ref, w_ref, b_ref, o_ref):
        agg = p_ref[0] + p_ref[1]
        o_ref[...] = (
            jnp.dot(agg, w_ref[...], preferred_element_type=jnp.float32)
            + b_ref[...]
        )

    return pl.pallas_call(
        body,
        grid=(N // BN,),
        in_specs=[
            pl.BlockSpec((2, BN, D), lambda i: (0, i, 0)),
            pl.BlockSpec((D, D), lambda i: (0, 0)),
            pl.BlockSpec((1, D), lambda i: (0, 0)),
        ],
        out_specs=pl.BlockSpec((BN, D), lambda i: (i, 0)),
        out_shape=jax.ShapeDtypeStruct((N, D), jnp.float32),
    )(parts, weight, bias2d)


@jax.jit
def kernel(feat, edge_index, eweight, weight, bias):
    parts = _sc_aggregate(edge_index, eweight.reshape(E), feat)
    return _tc_matmul(parts.reshape(2, N, D), weight, bias.reshape(1, D))


# speedup vs baseline: 1.0002x; 1.0002x over previous
import functools

import jax
import jax.numpy as jnp
from jax import lax
from jax.experimental import pallas as pl
from jax.experimental.pallas import tpu as pltpu
from jax.experimental.pallas import tpu_sc as plsc

N = 10000
E = 320000
D = 128
L = 16
NC = 2
NS = 16
NW = NC * NS
EPT = E // NW
C = 80
NCHUNK = EPT // C
ZR = C
ZCH = N // ZR
ZROUNDS = -(-ZCH // NS)
assert EPT % C == 0 and N % ZR == 0 and C % 8 == 0 and ZR % 8 == 0


def _sc_aggregate(src, ew, feat):
    mesh = plsc.VectorSubcoreMesh(
        core_axis_name="c", subcore_axis_name="s", num_cores=NC, num_subcores=NS
    )

    @functools.partial(
        pl.kernel,
        out_type=jax.ShapeDtypeStruct((NC * N, D), jnp.float32),
        mesh=mesh,
        scratch_types=[
            pltpu.VMEM((1, EPT), jnp.int32),
            pltpu.VMEM((1, EPT), jnp.int32),
            pltpu.VMEM((C, D), jnp.float32),
            pltpu.VMEM((C, D), jnp.float32),
            pltpu.VMEM((C, D), jnp.float32),
            pltpu.VMEM((C,), jnp.float32),
            pltpu.VMEM((C,), jnp.float32),
            pltpu.VMEM((C,), jnp.float32),
            pltpu.VMEM_SHARED((N, D), jnp.float32),
            [pltpu.SemaphoreType.DMA] * 3,
            [pltpu.SemaphoreType.DMA] * 3,
            [pltpu.SemaphoreType.DMA] * 3,
        ],
        compiler_params=pltpu.CompilerParams(use_tc_tiling_on_sc=False),
    )
    def body(edge_hbm, ew_hbm, feat_hbm, out_hbm,
             src_v, dst_v, rows0_v, rows1_v, rows2_v,
             ewb0_v, ewb1_v, ewb2_v, acc_sh, gsems, esems, ssems):
        stage_v = rows0_v
        c = lax.axis_index("c")
        s = lax.axis_index("s")
        wid = s * NC + c

        rows = (rows0_v, rows1_v, rows2_v)
        ewbs = (ewb0_v, ewb1_v, ewb2_v)

        pltpu.sync_copy(edge_hbm.at[pl.ds(0, 1), pl.ds(wid * EPT, EPT)], src_v)
        pltpu.sync_copy(edge_hbm.at[pl.ds(1, 1), pl.ds(wid * EPT, EPT)], dst_v)

        @plsc.parallel_loop(0, ZR, 1, unroll=4)
        def zero_row(i):
            for j in range(D // L):
                stage_v[i, pl.ds(j * L, L)] = jnp.zeros((L,), jnp.float32)

        for k in range(ZROUNDS):
            i = s + k * NS

            @pl.when(i < ZCH)
            def _():
                pltpu.async_copy(stage_v, acc_sh.at[pl.ds(i * ZR, ZR)],
                                 gsems[0])

        for k in range(ZROUNDS):
            i = s + k * NS

            @pl.when(i < ZCH)
            def _():
                pltpu.make_async_copy(
                    stage_v, acc_sh.at[pl.ds(i * ZR, ZR)], gsems[0]
                ).wait()

        plsc.subcore_barrier()

        dnums = lax.GatherDimensionNumbers(
            offset_dims=(), collapsed_slice_dims=(0,), start_index_map=(0,)
        )

        def gather_start(ci, b):
            pltpu.make_async_copy(
                feat_hbm.at[src_v.at[0, pl.ds(ci * C, C)]], rows[b], gsems[b]
            ).start()
            pltpu.async_copy(ew_hbm.at[pl.ds(wid * EPT + ci * C, C)],
                             ewbs[b], esems[b])

        def gather_wait(ci, b):
            pltpu.make_async_copy(
                feat_hbm.at[src_v.at[0, pl.ds(ci * C, C)]], rows[b], gsems[b]
            ).wait()
            pltpu.make_async_copy(
                ew_hbm.at[pl.ds(wid * EPT + ci * C, C)], ewbs[b], esems[b]
            ).wait()

        def scatter_start(ci, b):
            pltpu.async_copy(rows[b], acc_sh.at[dst_v.at[0, pl.ds(ci * C, C)]],
                             ssems[b], add=True)

        def scatter_wait(ci, b):
            pltpu.make_async_copy(
                rows[b], acc_sh.at[dst_v.at[0, pl.ds(ci * C, C)]], ssems[b]
            ).wait()

        def scale(ci, b):
            buf, ewb = rows[b], ewbs[b]

            @plsc.parallel_loop(0, C // L, 1, unroll=2)
            def group(g):
                wvec = ewb[pl.ds(g * L, L)]
                for ll in range(L):
                    e = g * L + ll
                    w = lax.gather(
                        wvec,
                        jnp.full((L, 1), ll, jnp.int32),
                        dnums,
                        slice_sizes=(1,),
                        mode=lax.GatherScatterMode.PROMISE_IN_BOUNDS,
                    )
                    for j in range(D // L):
                        sl = pl.ds(j * L, L)
                        buf[e, sl] = buf[e, sl] * w

        gather_start(0, 0)
        gather_start(1, 1)
        gather_wait(0, 0)
        scale(0, 0)
        gather_start(2, 2)
        scatter_start(0, 0)
        gather_wait(1, 1)
        scale(1, 1)
        scatter_start(1, 1)

        def chunk_triple(o, carry):
            c0 = 3 * o + 2
            scatter_wait(c0 - 2, 0)
            gather_start(c0 + 1, 0)
            gather_wait(c0, 2)
            scale(c0, 2)
            scatter_start(c0, 2)

            scatter_wait(c0 - 1, 1)
            gather_start(c0 + 2, 1)
            gather_wait(c0 + 1, 0)
            scale(c0 + 1, 0)
            scatter_start(c0 + 1, 0)

            scatter_wait(c0, 2)

            @pl.when(c0 + 3 < NCHUNK)
            def _():
                gather_start(c0 + 3, 2)

            gather_wait(c0 + 2, 1)
            scale(c0 + 2, 1)
            scatter_start(c0 + 2, 1)
            return carry

        lax.fori_loop(0, (NCHUNK - 2) // 3, chunk_triple, 0)
        scatter_wait(NCHUNK - 2, 0)
        scatter_wait(NCHUNK - 1, 1)
        plsc.subcore_barrier()

        for k in range(ZROUNDS):
            b = k % 3
            i = s + k * NS

            @pl.when(i < ZCH)
            def _():
                if k >= 3:
                    ip = s + (k - 3) * NS
                    pltpu.make_async_copy(
                        rows[b], out_hbm.at[pl.ds(c * N + ip * ZR, ZR)],
                        ssems[b],
                    ).wait()
                pltpu.sync_copy(acc_sh.at[pl.ds(i * ZR, ZR)], rows[b])
                pltpu.async_copy(
                    rows[b], out_hbm.at[pl.ds(c * N + i * ZR, ZR)], ssems[b]
                )

        for k in range(ZROUNDS - 3, ZROUNDS):
            b = k % 3
            i = s + k * NS

            @pl.when(i < ZCH)
            def _():
                pltpu.make_async_copy(
                    rows[b], out_hbm.at[pl.ds(c * N + i * ZR, ZR)], ssems[b]
                ).wait()

    return body(src, ew, feat)


BN = 1000


def _tc_matmul(parts, weight, bias2d):
    def body(p_ref, w_ref, b_ref, o_ref):
        agg = p_ref[0] + p_ref[1]
        o_ref[...] = (
            jnp.dot(agg, w_ref[...], preferred_element_type=jnp.float32)
            + b_ref[...]
        )

    return pl.pallas_call(
        body,
        grid=(N // BN,),
        in_specs=[
            pl.BlockSpec((2, BN, D), lambda i: (0, i, 0)),
            pl.BlockSpec((D, D), lambda i: (0, 0)),
            pl.BlockSpec((1, D), lambda i: (0, 0)),
        ],
        out_specs=pl.BlockSpec((BN, D), lambda i: (i, 0)),
        out_shape=jax.ShapeDtypeStruct((N, D), jnp.float32),
    )(parts, weight, bias2d)


@jax.jit
def kernel(feat, edge_index, eweight, weight, bias):
    parts = _sc_aggregate(edge_index, eweight[:, 0], feat)
    return _tc_matmul(parts.reshape(2, N, D), weight, bias.reshape(1, D))


# speedup vs baseline: 1.0015x; 1.0013x over previous
import functools

import jax
import jax.numpy as jnp
from jax import lax
from jax.experimental import pallas as pl
from jax.experimental.pallas import tpu as pltpu
from jax.experimental.pallas import tpu_sc as plsc

N = 10000
E = 320000
D = 128
L = 16
NC = 2
NS = 16
NW = NC * NS
EPT = E // NW
C = 80
NCHUNK = EPT // C
ZR = C
ZCH = N // ZR
ZROUNDS = -(-ZCH // NS)
assert EPT % C == 0 and N % ZR == 0 and C % 8 == 0 and ZR % 8 == 0


def _sc_aggregate(src, ew, feat):
    mesh = plsc.VectorSubcoreMesh(
        core_axis_name="c", subcore_axis_name="s", num_cores=NC, num_subcores=NS
    )

    @functools.partial(
        pl.kernel,
        out_type=jax.ShapeDtypeStruct((NC * N, D), jnp.float32),
        mesh=mesh,
        scratch_types=[
            pltpu.VMEM((NCHUNK, C), jnp.int32),
            pltpu.VMEM((NCHUNK, C), jnp.int32),
            pltpu.VMEM((C, D), jnp.float32),
            pltpu.VMEM((C, D), jnp.float32),
            pltpu.VMEM((C, D), jnp.float32),
            pltpu.VMEM((C,), jnp.float32),
            pltpu.VMEM((C,), jnp.float32),
            pltpu.VMEM((C,), jnp.float32),
            pltpu.VMEM_SHARED((N, D), jnp.float32),
            [pltpu.SemaphoreType.DMA] * 3,
            [pltpu.SemaphoreType.DMA] * 3,
            [pltpu.SemaphoreType.DMA] * 3,
        ],
        compiler_params=pltpu.CompilerParams(use_tc_tiling_on_sc=False),
    )
    def body(edge_hbm, ew_hbm, feat_hbm, out_hbm,
             src_v, dst_v, rows0_v, rows1_v, rows2_v,
             ewb0_v, ewb1_v, ewb2_v, acc_sh, gsems, esems, ssems):
        stage_v = rows0_v
        c = lax.axis_index("c")
        s = lax.axis_index("s")
        wid = s * NC + c

        rows = (rows0_v, rows1_v, rows2_v)
        ewbs = (ewb0_v, ewb1_v, ewb2_v)

        pltpu.sync_copy(edge_hbm.at[0, wid], src_v)
        pltpu.sync_copy(edge_hbm.at[1, wid], dst_v)

        @plsc.parallel_loop(0, ZR, 1, unroll=4)
        def zero_row(i):
            for j in range(D // L):
                stage_v[i, pl.ds(j * L, L)] = jnp.zeros((L,), jnp.float32)

        for k in range(ZROUNDS):
            i = s + k * NS

            @pl.when(i < ZCH)
            def _():
                pltpu.async_copy(stage_v, acc_sh.at[pl.ds(i * ZR, ZR)],
                                 gsems[0])

        for k in range(ZROUNDS):
            i = s + k * NS

            @pl.when(i < ZCH)
            def _():
                pltpu.make_async_copy(
                    stage_v, acc_sh.at[pl.ds(i * ZR, ZR)], gsems[0]
                ).wait()

        plsc.subcore_barrier()

        dnums = lax.GatherDimensionNumbers(
            offset_dims=(), collapsed_slice_dims=(0,), start_index_map=(0,)
        )

        def gather_start(ci, b):
            pltpu.make_async_copy(
                feat_hbm.at[src_v.at[ci]], rows[b], gsems[b]
            ).start()
            pltpu.async_copy(ew_hbm.at[pl.ds(wid * EPT + ci * C, C)],
                             ewbs[b], esems[b])

        def gather_wait(ci, b):
            pltpu.make_async_copy(
                feat_hbm.at[src_v.at[ci]], rows[b], gsems[b]
            ).wait()
            pltpu.make_async_copy(
                ew_hbm.at[pl.ds(wid * EPT + ci * C, C)], ewbs[b], esems[b]
            ).wait()

        def scatter_start(ci, b):
            pltpu.async_copy(rows[b], acc_sh.at[dst_v.at[ci]], ssems[b],
                             add=True)

        def scatter_wait(ci, b):
            pltpu.make_async_copy(
                rows[b], acc_sh.at[dst_v.at[ci]], ssems[b]
            ).wait()

        def scale(ci, b):
            buf, ewb = rows[b], ewbs[b]

            @plsc.parallel_loop(0, C // L, 1, unroll=2)
            def group(g):
                wvec = ewb[pl.ds(g * L, L)]
                for ll in range(L):
                    e = g * L + ll
                    w = lax.gather(
                        wvec,
                        jnp.full((L, 1), ll, jnp.int32),
                        dnums,
                        slice_sizes=(1,),
                        mode=lax.GatherScatterMode.PROMISE_IN_BOUNDS,
                    )
                    for j in range(D // L):
                        sl = pl.ds(j * L, L)
                        buf[e, sl] = buf[e, sl] * w

        gather_start(0, 0)
        gather_start(1, 1)
        gather_wait(0, 0)
        scale(0, 0)
        gather_start(2, 2)
        scatter_start(0, 0)
        gather_wait(1, 1)
        scale(1, 1)
        scatter_start(1, 1)

        def chunk_triple(o, carry):
            c0 = 3 * o + 2
            scatter_wait(c0 - 2, 0)
            gather_start(c0 + 1, 0)
            gather_wait(c0, 2)
            scale(c0, 2)
            scatter_start(c0, 2)

            scatter_wait(c0 - 1, 1)
            gather_start(c0 + 2, 1)
            gather_wait(c0 + 1, 0)
            scale(c0 + 1, 0)
            scatter_start(c0 + 1, 0)

            scatter_wait(c0, 2)

            @pl.when(c0 + 3 < NCHUNK)
            def _():
                gather_start(c0 + 3, 2)

            gather_wait(c0 + 2, 1)
            scale(c0 + 2, 1)
            scatter_start(c0 + 2, 1)
            return carry

        lax.fori_loop(0, (NCHUNK - 2) // 3, chunk_triple, 0)
        scatter_wait(NCHUNK - 2, 0)
        scatter_wait(NCHUNK - 1, 1)
        plsc.subcore_barrier()

        for k in range(ZROUNDS):
            b = k % 3
            i = s + k * NS

            @pl.when(i < ZCH)
            def _():
                if k >= 3:
                    ip = s + (k - 3) * NS
                    pltpu.make_async_copy(
                        rows[b], out_hbm.at[pl.ds(c * N + ip * ZR, ZR)],
                        ssems[b],
                    ).wait()
                pltpu.sync_copy(acc_sh.at[pl.ds(i * ZR, ZR)], rows[b])
                pltpu.async_copy(
                    rows[b], out_hbm.at[pl.ds(c * N + i * ZR, ZR)], ssems[b]
                )

        for k in range(ZROUNDS - 3, ZROUNDS):
            b = k % 3
            i = s + k * NS

            @pl.when(i < ZCH)
            def _():
                pltpu.make_async_copy(
                    rows[b], out_hbm.at[pl.ds(c * N + i * ZR, ZR)], ssems[b]
                ).wait()

    return body(src, ew, feat)


BN = 1000


def _tc_matmul(parts, weight, bias2d):
    def body(p_ref, w_ref, b_ref, o_ref):
        agg = p_ref[0] + p_ref[1]
        o_ref[...] = (
            jnp.dot(agg, w_ref[...], preferred_element_type=jnp.float32)
            + b_ref[...]
        )

    return pl.pallas_call(
        body,
        grid=(N // BN,),
        in_specs=[
            pl.BlockSpec((2, BN, D), lambda i: (0, i, 0)),
            pl.BlockSpec((D, D), lambda i: (0, 0)),
            pl.BlockSpec((1, D), lambda i: (0, 0)),
        ],
        out_specs=pl.BlockSpec((BN, D), lambda i: (i, 0)),
        out_shape=jax.ShapeDtypeStruct((N, D), jnp.float32),
    )(parts, weight, bias2d)


@jax.jit
def kernel(feat, edge_index, eweight, weight, bias):
    edges = edge_index.reshape(2, NW, NCHUNK, C)
    ew = eweight.reshape(E)
    parts = _sc_aggregate(edges, ew, feat)
    return _tc_matmul(parts.reshape(2, N, D), weight, bias.reshape(1, D))
